# baseline (speedup 1.0000x reference)
import functools

import jax
import jax.numpy as jnp
from jax import lax
from jax.experimental import pallas as pl
from jax.experimental.pallas import tpu as pltpu
from jax.experimental.pallas import tpu_sc as plsc

BATCH = 16384
N_DIM = 64
PAIR = 2 * N_DIM
L = 16
NC, NS = 2, 16
NW = NC * NS
CHUNK = 128
ROWS_PER_W = BATCH // NW // CHUNK
GROUPS = CHUNK // L
NBUF = 2


def _body(uh_hbm, up_hbm, u_hbm, ih_hbm, ip_hbm, i_hbm, a_hbm,
          uv_hbm, ub_hbm, iv_hbm, ib_hbm, c1_hbm, c0_hbm, out_hbm,
          uh_v, up_v, u_v, ih_v, ip_v, i_v, a_v, c1_v, c0_v,
          ub0, ub1, ib0, ib1, ubias, ibias, out_v,
          sems):
    ubuf = [ub0, ub1]
    ibuf = [ib0, ib1]
    wid = lax.axis_index("s") * NC + lax.axis_index("c")
    base = wid * ROWS_PER_W

    pltpu.sync_copy(uh_hbm.at[pl.ds(base, ROWS_PER_W)], uh_v)
    pltpu.sync_copy(up_hbm.at[pl.ds(base, ROWS_PER_W)], up_v)
    pltpu.sync_copy(u_hbm.at[pl.ds(base, ROWS_PER_W)], u_v)
    pltpu.sync_copy(ih_hbm.at[pl.ds(base, ROWS_PER_W)], ih_v)
    pltpu.sync_copy(ip_hbm.at[pl.ds(base, ROWS_PER_W)], ip_v)
    pltpu.sync_copy(i_hbm.at[pl.ds(base, ROWS_PER_W)], i_v)
    pltpu.sync_copy(a_hbm.at[pl.ds(base, ROWS_PER_W)], a_v)
    pltpu.sync_copy(c1_hbm, c1_v)
    pltpu.sync_copy(c0_hbm, c0_v)

    def fire(j):
        return (
            pltpu.async_copy(uv_hbm.at[uh_v.at[j]], ubuf[j % NBUF],
                             sems.at[j]),
            pltpu.async_copy(iv_hbm.at[ih_v.at[j]], ibuf[j % NBUF],
                             sems.at[j]),
            pltpu.async_copy(ub_hbm.at[u_v.at[j]], ubias.at[j], sems.at[j]),
            pltpu.async_copy(ib_hbm.at[i_v.at[j]], ibias.at[j], sems.at[j]),
        )

    descs = [fire(j) for j in range(NBUF)]

    c1v = c1_v[...]
    c0v = c0_v[...]
    lane = jnp.arange(L, dtype=jnp.int32)

    for j in range(ROWS_PER_W):
        for d in descs[j]:
            d.wait()
        ub_j, ib_j = ubuf[j % NBUF], ibuf[j % NBUF]

        def group(g, carry, j=j, ub_j=ub_j, ib_j=ib_j):
            sl = pl.ds(g * L, L)
            rows = lane + g * L
            pu = up_v[j, sl]
            pi = ip_v[j, sl]
            accs = [a_v[j, sl] * c1v + c0v,
                    ubias[j, sl] + ibias[j, sl],
                    jnp.zeros((L,), jnp.float32),
                    jnp.zeros((L,), jnp.float32)]
            cu = [pu + k for k in range(4)]
            ci = [pi + k for k in range(4)]
            for q in range(N_DIM // 4):
                for k in range(4):
                    xu = plsc.load_gather(ub_j, [rows, cu[k]])
                    xi = plsc.load_gather(ib_j, [rows, ci[k]])
                    accs[k] = accs[k] + xu * xi
                    cu[k] = cu[k] + 4
                    ci[k] = ci[k] + 4
            out_v[j, sl] = (accs[0] + accs[1]) + (accs[2] + accs[3])
            return carry

        lax.fori_loop(0, GROUPS, group, 0)
        if j + NBUF < ROWS_PER_W:
            descs.append(fire(j + NBUF))

    pltpu.sync_copy(out_v, out_hbm.at[pl.ds(base, ROWS_PER_W)])


@jax.jit
def _mfpoly2_sc(uh, up, u2, ih, ip, i2, a2,
                uv_pairs, ub_flat, iv_pairs, ib_flat, c1, c0):
    mesh = plsc.VectorSubcoreMesh(core_axis_name="c", subcore_axis_name="s")
    f = functools.partial(
        pl.kernel,
        mesh=mesh,
        compiler_params=pltpu.CompilerParams(needs_layout_passes=False),
        out_type=jax.ShapeDtypeStruct((BATCH // CHUNK, CHUNK), jnp.float32),
        scratch_types=[
            pltpu.VMEM((ROWS_PER_W, CHUNK), jnp.int32),
            pltpu.VMEM((ROWS_PER_W, CHUNK), jnp.int32),
            pltpu.VMEM((ROWS_PER_W, CHUNK), jnp.int32),
            pltpu.VMEM((ROWS_PER_W, CHUNK), jnp.int32),
            pltpu.VMEM((ROWS_PER_W, CHUNK), jnp.int32),
            pltpu.VMEM((ROWS_PER_W, CHUNK), jnp.int32),
            pltpu.VMEM((ROWS_PER_W, CHUNK), jnp.float32),
            pltpu.VMEM((L,), jnp.float32),
            pltpu.VMEM((L,), jnp.float32),
            pltpu.VMEM((CHUNK, PAIR), jnp.float32),
            pltpu.VMEM((CHUNK, PAIR), jnp.float32),
            pltpu.VMEM((CHUNK, PAIR), jnp.float32),
            pltpu.VMEM((CHUNK, PAIR), jnp.float32),
            pltpu.VMEM((ROWS_PER_W, CHUNK), jnp.float32),
            pltpu.VMEM((ROWS_PER_W, CHUNK), jnp.float32),
            pltpu.VMEM((ROWS_PER_W, CHUNK), jnp.float32),
            pltpu.SemaphoreType.DMA((ROWS_PER_W,)),
        ],
    )(_body)
    return f(uh, up, u2, ih, ip, i2, a2,
             uv_pairs, ub_flat, iv_pairs, ib_flat, c1, c0)


def kernel(u, i, a, user_vect, user_bias, item_vect, item_bias, glob_bias,
           age1_w, age1_b, age2_w, age2_b):
    n = u.shape[0]
    u32 = u.astype(jnp.int32)
    i32 = i.astype(jnp.int32)
    shp = (n // CHUNK, CHUNK)
    uh = (u32 >> 1).reshape(shp)
    up = ((u32 & 1) << 6).reshape(shp)
    u2 = u32.reshape(shp)
    ih = (i32 >> 1).reshape(shp)
    ip = ((i32 & 1) << 6).reshape(shp)
    i2 = i32.reshape(shp)
    a2 = a.reshape(shp)
    uv_pairs = user_vect.reshape(user_vect.shape[0] // 2, PAIR)
    iv_pairs = item_vect.reshape(item_vect.shape[0] // 2, PAIR)
    c1 = age1_w[0, 0] * age2_w[0, 0]
    c0 = glob_bias[0, 0] + age1_b[0] * age2_w[0, 0] + age2_b[0]
    c1v = jnp.full((L,), c1, jnp.float32)
    c0v = jnp.full((L,), c0, jnp.float32)
    out2 = _mfpoly2_sc(uh, up, u2, ih, ip, i2, a2,
                       uv_pairs, user_bias.reshape(-1),
                       iv_pairs, item_bias.reshape(-1), c1v, c0v)
    return out2.reshape(n)


# speedup vs baseline: 1.4701x; 1.4701x over previous
import functools

import jax
import jax.numpy as jnp
from jax import lax
from jax.experimental import pallas as pl
from jax.experimental.pallas import tpu as pltpu
from jax.experimental.pallas import tpu_sc as plsc

BATCH = 16384
N_DIM = 64
L = 16
NC, NS = 2, 16
NW = NC * NS
PER_W = BATCH // NW
GROUPS = PER_W // L
BCHUNK = 128


def _body(u_hbm, i_hbm, a_hbm, uv_hbm, ub_hbm, iv_hbm, ib_hbm,
          c1_hbm, c0_hbm, out_hbm,
          u_v, i_v, a_v, c1_v, c0_v,
          ubf0, ubf1, ibf0, ibf1, ubias, ibias, out_v,
          bsem, sems):
    ubf = [ubf0, ubf1]
    ibf = [ibf0, ibf1]
    wid = lax.axis_index("s") * NC + lax.axis_index("c")
    base = wid * PER_W

    pltpu.sync_copy(u_hbm.at[pl.ds(base, PER_W)], u_v)
    pltpu.sync_copy(i_hbm.at[pl.ds(base, PER_W)], i_v)
    pltpu.sync_copy(a_hbm.at[pl.ds(base, PER_W)], a_v)
    pltpu.sync_copy(c1_hbm, c1_v)
    pltpu.sync_copy(c0_hbm, c0_v)

    bias_descs = [
        pltpu.async_copy(t_hbm.at[t_v.at[pl.ds(c * BCHUNK, BCHUNK)]],
                         t_bias.at[pl.ds(c * BCHUNK, BCHUNK)], bsem)
        for c in range(PER_W // BCHUNK)
        for (t_hbm, t_v, t_bias) in ((ub_hbm, u_v, ubias),
                                     (ib_hbm, i_v, ibias))
    ]

    c1v = c1_v[...]
    c0v = c0_v[...]
    lane = jnp.arange(L, dtype=jnp.int32)

    def fire(g, par):
        uvals = u_v[pl.ds(g * L, L)]
        ivals = i_v[pl.ds(g * L, L)]
        for k in range(L):
            us = pl.multiple_of((uvals[k] >> 3) << 3, 8)
            pltpu.async_copy(uv_hbm.at[pl.ds(us, 8), :], ubf[par].at[k],
                             sems.at[par])
            is_ = pl.multiple_of((ivals[k] >> 3) << 3, 8)
            pltpu.async_copy(iv_hbm.at[pl.ds(is_, 8), :], ibf[par].at[k],
                             sems.at[par])

    def drain(par):
        for k in range(L):
            pltpu.make_async_copy(uv_hbm.at[pl.ds(0, 8), :],
                                  ubf[par].at[k], sems.at[par]).wait()
            pltpu.make_async_copy(uv_hbm.at[pl.ds(0, 8), :],
                                  ibf[par].at[k], sems.at[par]).wait()

    def compute(g, par):
        sl = pl.ds(g * L, L)
        usub = u_v[sl] & 7
        isub = i_v[sl] & 7
        accs = [a_v[sl] * c1v + c0v,
                ubias[sl] + ibias[sl],
                jnp.zeros((L,), jnp.float32),
                jnp.zeros((L,), jnp.float32)]
        dc = jnp.zeros((L,), jnp.int32)
        for d in range(N_DIM):
            xu = plsc.load_gather(ubf[par], [lane, usub, dc])
            xi = plsc.load_gather(ibf[par], [lane, isub, dc])
            accs[d % 4] = accs[d % 4] + xu * xi
            dc = dc + 1
        out_v[sl] = (accs[0] + accs[1]) + (accs[2] + accs[3])

    fire(0, 0)
    fire(1, 1)
    for d in bias_descs:
        d.wait()

    def pair(gp, carry):
        g0 = gp * 2
        drain(0)
        compute(g0, 0)

        @pl.when(g0 + 2 < GROUPS)
        def _():
            fire(g0 + 2, 0)

        drain(1)
        compute(g0 + 1, 1)

        @pl.when(g0 + 3 < GROUPS)
        def _():
            fire(g0 + 3, 1)

        return carry

    lax.fori_loop(0, GROUPS // 2, pair, 0)

    pltpu.sync_copy(out_v, out_hbm.at[pl.ds(base, PER_W)])


@jax.jit
def _mfpoly2_sc(u1, i1, a1, user_vect, ub_flat, item_vect, ib_flat, c1, c0):
    mesh = plsc.VectorSubcoreMesh(core_axis_name="c", subcore_axis_name="s")
    f = functools.partial(
        pl.kernel,
        mesh=mesh,
        compiler_params=pltpu.CompilerParams(needs_layout_passes=False),
        out_type=jax.ShapeDtypeStruct((BATCH,), jnp.float32),
        scratch_types=[
            pltpu.VMEM((PER_W,), jnp.int32),
            pltpu.VMEM((PER_W,), jnp.int32),
            pltpu.VMEM((PER_W,), jnp.float32),
            pltpu.VMEM((L,), jnp.float32),
            pltpu.VMEM((L,), jnp.float32),
            pltpu.VMEM((L, 8, N_DIM), jnp.float32),
            pltpu.VMEM((L, 8, N_DIM), jnp.float32),
            pltpu.VMEM((L, 8, N_DIM), jnp.float32),
            pltpu.VMEM((L, 8, N_DIM), jnp.float32),
            pltpu.VMEM((PER_W,), jnp.float32),
            pltpu.VMEM((PER_W,), jnp.float32),
            pltpu.VMEM((PER_W,), jnp.float32),
            pltpu.SemaphoreType.DMA,
            pltpu.SemaphoreType.DMA((2,)),
        ],
    )(_body)
    return f(u1, i1, a1, user_vect, ub_flat, item_vect, ib_flat, c1, c0)


def kernel(u, i, a, user_vect, user_bias, item_vect, item_bias, glob_bias,
           age1_w, age1_b, age2_w, age2_b):
    n = u.shape[0]
    c1 = age1_w[0, 0] * age2_w[0, 0]
    c0 = glob_bias[0, 0] + age1_b[0] * age2_w[0, 0] + age2_b[0]
    c1v = jnp.full((L,), c1, jnp.float32)
    c0v = jnp.full((L,), c0, jnp.float32)
    return _mfpoly2_sc(u.astype(jnp.int32), i.astype(jnp.int32), a,
                       user_vect, user_bias.reshape(-1),
                       item_vect, item_bias.reshape(-1), c1v, c0v)
